# Initial kernel scaffold; baseline (speedup 1.0000x reference)
import functools

import jax
import jax.numpy as jnp
from jax import lax
from jax.experimental import pallas as pl
from jax.experimental.pallas import tpu as pltpu
from jax.experimental.pallas import tpu_sc as plsc

_B, _L, _NUM_PIDS = 128, 4096, 256
_EPS = 1e-06
_LANES = 16
_NC, _NS = 2, 16
_NW = _NC * _NS
_ROWS_PER_W = _B // _NW
_CHUNKS = _L // _LANES
_PID_CHUNKS = _NUM_PIDS // _LANES


def _sc_partials(yhat, y, pm):
    mesh = plsc.VectorSubcoreMesh(core_axis_name="c", subcore_axis_name="s")

    @functools.partial(
        pl.kernel,
        mesh=mesh,
        out_type=jax.ShapeDtypeStruct((_NC, _LANES), jnp.float32),
        scratch_types=[
            pltpu.VMEM((_L,), jnp.float32),
            pltpu.VMEM((_L,), jnp.float32),
            pltpu.VMEM((_L,), jnp.int32),
            pltpu.VMEM((_NUM_PIDS,), jnp.float32),
            pltpu.VMEM((_NUM_PIDS,), jnp.float32),
            pltpu.VMEM((_LANES,), jnp.float32),
            pltpu.VMEM((_LANES,), jnp.float32),
            pltpu.VMEM((_NS * _LANES,), jnp.float32),
            pltpu.VMEM((_NS * _LANES,), jnp.float32),
            pltpu.VMEM_SHARED((_NS * _LANES,), jnp.float32),
            pltpu.VMEM_SHARED((_NS * _LANES,), jnp.float32),
        ],
    )
    def k(yhat_hbm, y_hbm, pm_hbm, out_hbm,
          yh_v, yy_v, pm_v, pred_t, true_t, st_sq, st_cnt,
          all_sq, all_cnt, sh_sq, sh_cnt):
        cid = lax.axis_index("c")
        sid = lax.axis_index("s")
        wid = cid * _NS + sid

        zero16 = jnp.zeros((_LANES,), jnp.float32)

        acc_sq = zero16
        acc_cnt = zero16
        for r in range(_ROWS_PER_W):
            row = wid * _ROWS_PER_W + r
            pltpu.sync_copy(yhat_hbm.at[row], yh_v)
            pltpu.sync_copy(y_hbm.at[row], yy_v)
            pltpu.sync_copy(pm_hbm.at[row], pm_v)

            def zbody(i, _):
                pred_t[pl.ds(i * _LANES, _LANES)] = zero16
                true_t[pl.ds(i * _LANES, _LANES)] = zero16
                return 0

            lax.fori_loop(0, _PID_CHUNKS, zbody, 0, unroll=4)

            def sbody(i, _):
                b = i * _LANES
                ph = pm_v[pl.ds(b, _LANES)]
                plsc.addupdate_scatter(pred_t, [ph], yh_v[pl.ds(b, _LANES)])
                plsc.addupdate_scatter(true_t, [ph], yy_v[pl.ds(b, _LANES)])
                return 0

            lax.fori_loop(0, _CHUNKS, sbody, 0, unroll=4)

            def pbody(i, carry):
                a_sq, a_cnt = carry
                b = i * _LANES
                ps = pred_t[pl.ds(b, _LANES)]
                ts = true_t[pl.ds(b, _LANES)]
                valid = ts > 0.0
                diff = jnp.where(valid, ps - ts, 0.0)
                return (a_sq + diff * diff,
                        a_cnt + jnp.where(valid, 1.0, 0.0))

            acc_sq, acc_cnt = lax.fori_loop(
                0, _PID_CHUNKS, pbody, (acc_sq, acc_cnt), unroll=4)

        st_sq[...] = acc_sq
        st_cnt[...] = acc_cnt
        pltpu.sync_copy(st_sq, sh_sq.at[pl.ds(sid * _LANES, _LANES)])
        pltpu.sync_copy(st_cnt, sh_cnt.at[pl.ds(sid * _LANES, _LANES)])
        plsc.subcore_barrier()

        @pl.when(sid == 0)
        def _():
            pltpu.sync_copy(sh_sq, all_sq)
            pltpu.sync_copy(sh_cnt, all_cnt)

            def rbody(i, carry):
                a_sq, a_cnt = carry
                b = i * _LANES
                return (a_sq + all_sq[pl.ds(b, _LANES)],
                        a_cnt + all_cnt[pl.ds(b, _LANES)])

            v_sq, v_cnt = lax.fori_loop(0, _NS, rbody, (zero16, zero16),
                                        unroll=4)
            lanes = lax.broadcasted_iota(jnp.int32, (_LANES,), 0)
            out_vec = (jnp.where(lanes == 0, jnp.sum(v_sq), 0.0)
                       + jnp.where(lanes == 1, jnp.sum(v_cnt), 0.0))
            st_sq[...] = out_vec
            pltpu.sync_copy(st_sq, out_hbm.at[cid])

    return k(yhat, y, pm)


def kernel(yhat, y, plot_mask):
    yhat = jnp.squeeze(yhat).astype(jnp.float32)
    y = jnp.squeeze(y).astype(jnp.float32)
    pm = jnp.squeeze(plot_mask).astype(jnp.int32)
    parts = _sc_partials(yhat, y, pm)
    total_sq = parts[0, 0] + parts[1, 0]
    total_cnt = jnp.maximum(parts[0, 1] + parts[1, 1], 1.0)
    return jnp.sqrt(total_sq / total_cnt + _EPS)


# speedup vs baseline: 18.2804x; 18.2804x over previous
import functools

import jax
import jax.numpy as jnp
from jax import lax
from jax.experimental import pallas as pl
from jax.experimental.pallas import tpu as pltpu
from jax.experimental.pallas import tpu_sc as plsc

_B, _L, _NUM_PIDS = 128, 4096, 256
_EPS = 1e-06
_LANES = 16
_NC, _NS = 2, 16
_NW = _NC * _NS
_ROWS_PER_W = _B // _NW
_CHUNKS = _L // _LANES
_PID_CHUNKS = _NUM_PIDS // _LANES


def _sc_partials(yhat, y, pm):
    mesh = plsc.VectorSubcoreMesh(core_axis_name="c", subcore_axis_name="s")

    @functools.partial(
        pl.kernel,
        mesh=mesh,
        out_type=jax.ShapeDtypeStruct((_NC, _LANES), jnp.float32),
        compiler_params=pltpu.CompilerParams(needs_layout_passes=False),
        scratch_types=[
            pltpu.VMEM((_L,), jnp.float32),
            pltpu.VMEM((_L,), jnp.float32),
            pltpu.VMEM((_L,), jnp.int32),
            pltpu.VMEM((_NUM_PIDS,), jnp.float32),
            pltpu.VMEM((_NUM_PIDS,), jnp.float32),
            pltpu.VMEM((_LANES,), jnp.float32),
            pltpu.VMEM((_LANES,), jnp.float32),
            pltpu.VMEM((_NS * _LANES,), jnp.float32),
            pltpu.VMEM((_NS * _LANES,), jnp.float32),
            pltpu.VMEM_SHARED((_NS * _LANES,), jnp.float32),
            pltpu.VMEM_SHARED((_NS * _LANES,), jnp.float32),
        ],
    )
    def k(yhat_hbm, y_hbm, pm_hbm, out_hbm,
          yh_v, yy_v, pm_v, pred_t, true_t, st_sq, st_cnt,
          all_sq, all_cnt, sh_sq, sh_cnt):
        cid = lax.axis_index("c")
        sid = lax.axis_index("s")
        wid = cid * _NS + sid

        zero16 = jnp.zeros((_LANES,), jnp.float32)

        acc_sq = zero16
        acc_cnt = zero16
        for r in range(_ROWS_PER_W):
            row = wid * _ROWS_PER_W + r
            pltpu.sync_copy(yhat_hbm.at[row], yh_v)
            pltpu.sync_copy(y_hbm.at[row], yy_v)
            pltpu.sync_copy(pm_hbm.at[row], pm_v)

            def zbody(i, _):
                pred_t[pl.ds(i * _LANES, _LANES)] = zero16
                true_t[pl.ds(i * _LANES, _LANES)] = zero16
                return 0

            lax.fori_loop(0, _PID_CHUNKS, zbody, 0, unroll=4)

            def sbody(i, _):
                b = i * _LANES
                ph = pm_v[pl.ds(b, _LANES)]
                plsc.addupdate_scatter(pred_t, [ph], yh_v[pl.ds(b, _LANES)])
                plsc.addupdate_scatter(true_t, [ph], yy_v[pl.ds(b, _LANES)])
                return 0

            lax.fori_loop(0, _CHUNKS, sbody, 0, unroll=4)

            def pbody(i, carry):
                a_sq, a_cnt = carry
                b = i * _LANES
                ps = pred_t[pl.ds(b, _LANES)]
                ts = true_t[pl.ds(b, _LANES)]
                valid = ts > 0.0
                diff = jnp.where(valid, ps - ts, 0.0)
                return (a_sq + diff * diff,
                        a_cnt + jnp.where(valid, 1.0, 0.0))

            acc_sq, acc_cnt = lax.fori_loop(
                0, _PID_CHUNKS, pbody, (acc_sq, acc_cnt), unroll=4)

        st_sq[...] = acc_sq
        st_cnt[...] = acc_cnt
        pltpu.sync_copy(st_sq, sh_sq.at[pl.ds(sid * _LANES, _LANES)])
        pltpu.sync_copy(st_cnt, sh_cnt.at[pl.ds(sid * _LANES, _LANES)])
        plsc.subcore_barrier()

        @pl.when(sid == 0)
        def _():
            pltpu.sync_copy(sh_sq, all_sq)
            pltpu.sync_copy(sh_cnt, all_cnt)

            def rbody(i, carry):
                a_sq, a_cnt = carry
                b = i * _LANES
                return (a_sq + all_sq[pl.ds(b, _LANES)],
                        a_cnt + all_cnt[pl.ds(b, _LANES)])

            v_sq, v_cnt = lax.fori_loop(0, _NS, rbody, (zero16, zero16),
                                        unroll=4)
            lanes = lax.broadcasted_iota(jnp.int32, (_LANES,), 0)
            out_vec = (jnp.where(lanes == 0, jnp.sum(v_sq), 0.0)
                       + jnp.where(lanes == 1, jnp.sum(v_cnt), 0.0))
            st_sq[...] = out_vec
            pltpu.sync_copy(st_sq, out_hbm.at[cid])

    return k(yhat, y, pm)


def kernel(yhat, y, plot_mask):
    yhat = jnp.squeeze(yhat).astype(jnp.float32)
    y = jnp.squeeze(y).astype(jnp.float32)
    pm = jnp.squeeze(plot_mask).astype(jnp.int32)
    parts = _sc_partials(yhat, y, pm)
    total_sq = parts[0, 0] + parts[1, 0]
    total_cnt = jnp.maximum(parts[0, 1] + parts[1, 1], 1.0)
    return jnp.sqrt(total_sq / total_cnt + _EPS)


# speedup vs baseline: 21.4693x; 1.1744x over previous
import functools

import jax
import jax.numpy as jnp
from jax import lax
from jax.experimental import pallas as pl
from jax.experimental.pallas import tpu as pltpu
from jax.experimental.pallas import tpu_sc as plsc

_B, _L, _NUM_PIDS = 128, 4096, 256
_EPS = 1e-06
_LANES = 16
_NC, _NS = 2, 16
_NW = _NC * _NS
_ROWS_PER_W = _B // _NW
_CHUNKS = _L // _LANES
_PID_CHUNKS = _NUM_PIDS // _LANES


def _sc_partials(yhat, y, pm):
    mesh = plsc.VectorSubcoreMesh(core_axis_name="c", subcore_axis_name="s")

    @functools.partial(
        pl.kernel,
        mesh=mesh,
        out_type=jax.ShapeDtypeStruct((_NC, _LANES), jnp.float32),
        compiler_params=pltpu.CompilerParams(needs_layout_passes=False),
        scratch_types=[
            pltpu.VMEM((_ROWS_PER_W, _L), jnp.float32),
            pltpu.VMEM((_ROWS_PER_W, _L), jnp.float32),
            pltpu.VMEM((_ROWS_PER_W, _L), jnp.int32),
            pltpu.VMEM((_NUM_PIDS,), jnp.float32),
            pltpu.VMEM((_NUM_PIDS,), jnp.float32),
            pltpu.VMEM((_LANES,), jnp.float32),
            pltpu.VMEM((_LANES,), jnp.float32),
            pltpu.VMEM((_NS * _LANES,), jnp.float32),
            pltpu.VMEM((_NS * _LANES,), jnp.float32),
            pltpu.VMEM_SHARED((_NS * _LANES,), jnp.float32),
            pltpu.VMEM_SHARED((_NS * _LANES,), jnp.float32),
            pltpu.SemaphoreType.DMA,
        ],
    )
    def k(yhat_hbm, y_hbm, pm_hbm, out_hbm,
          yh_v, yy_v, pm_v, pred_t, true_t, st_sq, st_cnt,
          all_sq, all_cnt, sh_sq, sh_cnt, sem):
        cid = lax.axis_index("c")
        sid = lax.axis_index("s")
        wid = cid * _NS + sid
        row0 = wid * _ROWS_PER_W

        zero16 = jnp.zeros((_LANES,), jnp.float32)

        cp_yh = pltpu.async_copy(yhat_hbm.at[pl.ds(row0, _ROWS_PER_W)],
                                 yh_v, sem)
        cp_yy = pltpu.async_copy(y_hbm.at[pl.ds(row0, _ROWS_PER_W)],
                                 yy_v, sem)
        cp_pm = pltpu.async_copy(pm_hbm.at[pl.ds(row0, _ROWS_PER_W)],
                                 pm_v, sem)

        def zbody(i, _):
            pred_t[pl.ds(i * _LANES, _LANES)] = zero16
            true_t[pl.ds(i * _LANES, _LANES)] = zero16
            return 0

        lax.fori_loop(0, _PID_CHUNKS, zbody, 0, unroll=4)

        cp_yh.wait()
        cp_yy.wait()
        cp_pm.wait()

        acc_sq = zero16
        acc_cnt = zero16
        for r in range(_ROWS_PER_W):

            def sbody(i, _):
                b = i * _LANES
                ph = pm_v[r, pl.ds(b, _LANES)]
                plsc.addupdate_scatter(pred_t, [ph], yh_v[r, pl.ds(b, _LANES)])
                plsc.addupdate_scatter(true_t, [ph], yy_v[r, pl.ds(b, _LANES)])
                return 0

            lax.fori_loop(0, _CHUNKS, sbody, 0, unroll=8)

            def pbody(i, carry):
                a_sq, a_cnt = carry
                b = i * _LANES
                ps = pred_t[pl.ds(b, _LANES)]
                ts = true_t[pl.ds(b, _LANES)]
                pred_t[pl.ds(b, _LANES)] = zero16
                true_t[pl.ds(b, _LANES)] = zero16
                valid = ts > 0.0
                diff = jnp.where(valid, ps - ts, 0.0)
                return (a_sq + diff * diff,
                        a_cnt + jnp.where(valid, 1.0, 0.0))

            acc_sq, acc_cnt = lax.fori_loop(
                0, _PID_CHUNKS, pbody, (acc_sq, acc_cnt), unroll=4)

        st_sq[...] = acc_sq
        st_cnt[...] = acc_cnt
        pltpu.sync_copy(st_sq, sh_sq.at[pl.ds(sid * _LANES, _LANES)])
        pltpu.sync_copy(st_cnt, sh_cnt.at[pl.ds(sid * _LANES, _LANES)])
        plsc.subcore_barrier()

        @pl.when(sid == 0)
        def _():
            pltpu.sync_copy(sh_sq, all_sq)
            pltpu.sync_copy(sh_cnt, all_cnt)

            def rbody(i, carry):
                a_sq, a_cnt = carry
                b = i * _LANES
                return (a_sq + all_sq[pl.ds(b, _LANES)],
                        a_cnt + all_cnt[pl.ds(b, _LANES)])

            v_sq, v_cnt = lax.fori_loop(0, _NS, rbody, (zero16, zero16),
                                        unroll=4)
            lanes = lax.broadcasted_iota(jnp.int32, (_LANES,), 0)
            out_vec = (jnp.where(lanes == 0, jnp.sum(v_sq), 0.0)
                       + jnp.where(lanes == 1, jnp.sum(v_cnt), 0.0))
            st_sq[...] = out_vec
            pltpu.sync_copy(st_sq, out_hbm.at[cid])

    return k(yhat, y, pm)


def kernel(yhat, y, plot_mask):
    yhat = jnp.squeeze(yhat).astype(jnp.float32)
    y = jnp.squeeze(y).astype(jnp.float32)
    pm = jnp.squeeze(plot_mask).astype(jnp.int32)
    parts = _sc_partials(yhat, y, pm)
    total_sq = parts[0, 0] + parts[1, 0]
    total_cnt = jnp.maximum(parts[0, 1] + parts[1, 1], 1.0)
    return jnp.sqrt(total_sq / total_cnt + _EPS)


# speedup vs baseline: 25.3535x; 1.1809x over previous
import functools

import jax
import jax.numpy as jnp
from jax import lax
from jax.experimental import pallas as pl
from jax.experimental.pallas import tpu as pltpu
from jax.experimental.pallas import tpu_sc as plsc

_B, _L, _NUM_PIDS = 128, 4096, 256
_EPS = 1e-06
_LANES = 16
_NC, _NS = 2, 16
_NW = _NC * _NS
_ROWS_PER_W = _B // _NW
_CHUNKS = _L // _LANES
_PID_CHUNKS = _NUM_PIDS // _LANES


def _sc_partials(yhat, y, pm):
    mesh = plsc.VectorSubcoreMesh(core_axis_name="c", subcore_axis_name="s")

    @functools.partial(
        pl.kernel,
        mesh=mesh,
        out_type=jax.ShapeDtypeStruct((_NC, _LANES), jnp.float32),
        compiler_params=pltpu.CompilerParams(needs_layout_passes=False),
        scratch_types=[
            pltpu.VMEM((_ROWS_PER_W, _L), jnp.float32),
            pltpu.VMEM((_ROWS_PER_W, _L), jnp.float32),
            pltpu.VMEM((_ROWS_PER_W, _L), jnp.int32),
            pltpu.VMEM((_NUM_PIDS,), jnp.float32),
            pltpu.VMEM((_NUM_PIDS,), jnp.float32),
            pltpu.VMEM((_LANES,), jnp.float32),
            pltpu.VMEM((_LANES,), jnp.float32),
            pltpu.VMEM((_NS * _LANES,), jnp.float32),
            pltpu.VMEM((_NS * _LANES,), jnp.float32),
            pltpu.VMEM_SHARED((_NS * _LANES,), jnp.float32),
            pltpu.VMEM_SHARED((_NS * _LANES,), jnp.float32),
            pltpu.SemaphoreType.DMA,
        ],
    )
    def k(yhat_hbm, y_hbm, pm_hbm, out_hbm,
          yh_v, yy_v, pm_v, pred_t, true_t, st_sq, st_cnt,
          all_sq, all_cnt, sh_sq, sh_cnt, sem):
        cid = lax.axis_index("c")
        sid = lax.axis_index("s")
        wid = cid * _NS + sid
        row0 = wid * _ROWS_PER_W

        zero16 = jnp.zeros((_LANES,), jnp.float32)

        cp_yh = pltpu.async_copy(yhat_hbm.at[pl.ds(row0, _ROWS_PER_W)],
                                 yh_v, sem)
        cp_yy = pltpu.async_copy(y_hbm.at[pl.ds(row0, _ROWS_PER_W)],
                                 yy_v, sem)
        cp_pm = pltpu.async_copy(pm_hbm.at[pl.ds(row0, _ROWS_PER_W)],
                                 pm_v, sem)

        def zbody(i, _):
            pred_t[pl.ds(i * _LANES, _LANES)] = zero16
            true_t[pl.ds(i * _LANES, _LANES)] = zero16
            return 0

        lax.fori_loop(0, _PID_CHUNKS, zbody, 0, unroll=4)

        cp_yh.wait()
        cp_yy.wait()
        cp_pm.wait()

        acc_sq = zero16
        acc_cnt = zero16
        for r in range(_ROWS_PER_W):

            @plsc.parallel_loop(0, _CHUNKS, unroll=8)
            def _(i):
                b = i * _LANES
                ph = pm_v[r, pl.ds(b, _LANES)]
                plsc.addupdate_scatter(pred_t, [ph], yh_v[r, pl.ds(b, _LANES)])
                plsc.addupdate_scatter(true_t, [ph], yy_v[r, pl.ds(b, _LANES)])

            def pbody(i, carry):
                a_sq, a_cnt = carry
                b = i * _LANES
                ps = pred_t[pl.ds(b, _LANES)]
                ts = true_t[pl.ds(b, _LANES)]
                pred_t[pl.ds(b, _LANES)] = zero16
                true_t[pl.ds(b, _LANES)] = zero16
                valid = ts > 0.0
                diff = jnp.where(valid, ps - ts, 0.0)
                return (a_sq + diff * diff,
                        a_cnt + jnp.where(valid, 1.0, 0.0))

            acc_sq, acc_cnt = lax.fori_loop(
                0, _PID_CHUNKS, pbody, (acc_sq, acc_cnt), unroll=4)

        st_sq[...] = acc_sq
        st_cnt[...] = acc_cnt
        pltpu.sync_copy(st_sq, sh_sq.at[pl.ds(sid * _LANES, _LANES)])
        pltpu.sync_copy(st_cnt, sh_cnt.at[pl.ds(sid * _LANES, _LANES)])
        plsc.subcore_barrier()

        @pl.when(sid == 0)
        def _():
            pltpu.sync_copy(sh_sq, all_sq)
            pltpu.sync_copy(sh_cnt, all_cnt)

            def rbody(i, carry):
                a_sq, a_cnt = carry
                b = i * _LANES
                return (a_sq + all_sq[pl.ds(b, _LANES)],
                        a_cnt + all_cnt[pl.ds(b, _LANES)])

            v_sq, v_cnt = lax.fori_loop(0, _NS, rbody, (zero16, zero16),
                                        unroll=4)
            lanes = lax.broadcasted_iota(jnp.int32, (_LANES,), 0)
            out_vec = (jnp.where(lanes == 0, jnp.sum(v_sq), 0.0)
                       + jnp.where(lanes == 1, jnp.sum(v_cnt), 0.0))
            st_sq[...] = out_vec
            pltpu.sync_copy(st_sq, out_hbm.at[cid])

    return k(yhat, y, pm)


def kernel(yhat, y, plot_mask):
    yhat = jnp.squeeze(yhat).astype(jnp.float32)
    y = jnp.squeeze(y).astype(jnp.float32)
    pm = jnp.squeeze(plot_mask).astype(jnp.int32)
    parts = _sc_partials(yhat, y, pm)
    total_sq = parts[0, 0] + parts[1, 0]
    total_cnt = jnp.maximum(parts[0, 1] + parts[1, 1], 1.0)
    return jnp.sqrt(total_sq / total_cnt + _EPS)
